# Initial kernel scaffold; baseline (speedup 1.0000x reference)
import jax
import jax.numpy as jnp
from jax.experimental import pallas as pl


def kernel(x0, x1, params):
    raise NotImplementedError("write your pallas kernel here")


# speedup vs baseline: 1.5104x; 1.5104x over previous
import functools

import jax
import jax.numpy as jnp
import numpy as np
from jax import lax
from jax.experimental import pallas as pl
from jax.experimental.pallas import tpu as pltpu
from jax.experimental.pallas import tpu_sc as plsc

SEQ = 2048
PKM_H = 4
PKM_KDIM = 128
PKM_KNN = 32
NEG = -3.0e38

_CAND_NJ = tuple(32 // (i + 1) for i in range(32))


def _mm_body(x_ref, w_ref, b_ref, o_ref, *, relu):
    acc = lax.dot_general(x_ref[...], w_ref[...], (((1,), (1,)), ((), ())),
                          preferred_element_type=jnp.float32)
    acc = acc + b_ref[...]
    if relu:
        acc = jnp.maximum(acc, 0.0)
    o_ref[...] = acc


def _matmul(x, W, b, relu=False, bm=256, bn=512):
    M, K = x.shape
    N = W.shape[0]
    bn = min(bn, N)
    return pl.pallas_call(
        functools.partial(_mm_body, relu=relu),
        grid=(M // bm, N // bn),
        in_specs=[
            pl.BlockSpec((bm, K), lambda i, j: (i, 0)),
            pl.BlockSpec((bn, K), lambda i, j: (j, 0)),
            pl.BlockSpec((1, bn), lambda i, j: (0, j)),
        ],
        out_specs=pl.BlockSpec((bm, bn), lambda i, j: (i, j)),
        out_shape=jax.ShapeDtypeStruct((M, N), jnp.float32),
    )(x, W, b.reshape(1, N))


def _attn_body(q_ref, k_ref, v_ref, o_ref, *, scale):
    q = q_ref[0]
    k = k_ref[0]
    s = lax.dot_general(q, k, (((1,), (1,)), ((), ())),
                        preferred_element_type=jnp.float32) * scale
    m = jnp.max(s, axis=-1, keepdims=True)
    e = jnp.exp(s - m)
    p = e / jnp.sum(e, axis=-1, keepdims=True)
    o_ref[0] = lax.dot_general(p, v_ref[0], (((1,), (0,)), ((), ())),
                               preferred_element_type=jnp.float32)


def _attention(q, k, v, bq=256):
    H, S, hd = q.shape
    return pl.pallas_call(
        functools.partial(_attn_body, scale=1.0 / np.sqrt(hd)),
        grid=(H, S // bq),
        in_specs=[
            pl.BlockSpec((1, bq, hd), lambda h, i: (h, i, 0)),
            pl.BlockSpec((1, S, hd), lambda h, i: (h, 0, 0)),
            pl.BlockSpec((1, S, hd), lambda h, i: (h, 0, 0)),
        ],
        out_specs=pl.BlockSpec((1, bq, hd), lambda h, i: (h, i, 0)),
        out_shape=jax.ShapeDtypeStruct((H, S, hd), jnp.float32),
    )(q, k, v)


def _ln_math(x, g_ref, b_ref, o_ref):
    mu = jnp.mean(x, axis=-1, keepdims=True)
    var = jnp.mean((x - mu) ** 2, axis=-1, keepdims=True)
    o_ref[...] = (x - mu) / jnp.sqrt(var + 1e-5) * g_ref[...] + b_ref[...]


def _ln_body(x_ref, y_ref, g_ref, b_ref, o_ref):
    _ln_math(x_ref[...] + y_ref[...], g_ref, b_ref, o_ref)


def _ln_body_nores(x_ref, g_ref, b_ref, o_ref):
    _ln_math(x_ref[...], g_ref, b_ref, o_ref)


def _res_ln(x, y, g, b, bm=256):
    M, D = x.shape
    has_y = y is not None
    body = _ln_body if has_y else _ln_body_nores
    specs = [pl.BlockSpec((bm, D), lambda i: (i, 0))]
    args = [x]
    if has_y:
        specs.append(pl.BlockSpec((bm, D), lambda i: (i, 0)))
        args.append(y)
    specs += [pl.BlockSpec((1, D), lambda i: (0, 0)),
              pl.BlockSpec((1, D), lambda i: (0, 0))]
    args += [g.reshape(1, D), b.reshape(1, D)]
    return pl.pallas_call(
        body,
        grid=(M // bm,),
        in_specs=specs,
        out_specs=pl.BlockSpec((bm, D), lambda i: (i, 0)),
        out_shape=jax.ShapeDtypeStruct((M, D), jnp.float32),
    )(*args)


def _topk_desc(s, n, kk):
    iota = lax.broadcasted_iota(jnp.int32, s.shape, 1)
    vals, idxs = [], []
    for _ in range(kk):
        m = jnp.max(s, axis=-1, keepdims=True)
        eq = s >= m
        ii = jnp.min(jnp.where(eq, iota, n), axis=-1, keepdims=True)
        vals.append(m)
        idxs.append(ii)
        s = jnp.where(eq, NEG, s)
    return jnp.concatenate(vals, axis=1), jnp.concatenate(idxs, axis=1)


def _pkm_score_body(q_ref, k1_ref, k2_ref, w_ref, i_ref, *, nk):
    q1 = q_ref[:, :PKM_KDIM // 2]
    q2 = q_ref[:, PKM_KDIM // 2:]
    dn = (((1,), (1,)), ((), ()))
    s1 = lax.dot_general(q1, k1_ref[0], dn, preferred_element_type=jnp.float32)
    s2 = lax.dot_general(q2, k2_ref[0], dn, preferred_element_type=jnp.float32)
    ts1, ti1 = _topk_desc(s1, nk, PKM_KNN)
    ts2, ti2 = _topk_desc(s2, nk, PKM_KNN)
    cv, ci = [], []
    for i, nj in enumerate(_CAND_NJ):
        cv.append(ts1[:, i:i + 1] + ts2[:, :nj])
        ci.append(ti1[:, i:i + 1] * nk + ti2[:, :nj])
    cvals = jnp.concatenate(cv, axis=1)
    cidx = jnp.concatenate(ci, axis=1)
    ncand = cvals.shape[1]
    iota = lax.broadcasted_iota(jnp.int32, cvals.shape, 1)
    vs, js = [], []
    s = cvals
    for _ in range(PKM_KNN):
        m = jnp.max(s, axis=-1, keepdims=True)
        eq = s >= m
        pos = jnp.min(jnp.where(eq, iota, ncand), axis=-1, keepdims=True)
        vs.append(m)
        js.append(jnp.max(jnp.where(iota == pos, cidx, -1), axis=-1,
                          keepdims=True))
        s = jnp.where(iota == pos, NEG, s)
    sc = jnp.concatenate(vs, axis=1)
    si = jnp.concatenate(js, axis=1)
    e = jnp.exp(sc - sc[:, :1])
    w_ref[0] = e / jnp.sum(e, axis=-1, keepdims=True)
    i_ref[0] = si


def _pkm_score(q, keys, nk, bt=256):
    S = q.shape[0]
    k1 = keys[:, 0]
    k2 = keys[:, 1]
    w, i = pl.pallas_call(
        functools.partial(_pkm_score_body, nk=nk),
        grid=(PKM_H, S // bt),
        in_specs=[
            pl.BlockSpec((bt, PKM_KDIM), lambda h, i: (i, h)),
            pl.BlockSpec((1, nk, PKM_KDIM // 2), lambda h, i: (h, 0, 0)),
            pl.BlockSpec((1, nk, PKM_KDIM // 2), lambda h, i: (h, 0, 0)),
        ],
        out_specs=[
            pl.BlockSpec((1, bt, PKM_KNN), lambda h, i: (h, i, 0)),
            pl.BlockSpec((1, bt, PKM_KNN), lambda h, i: (h, i, 0)),
        ],
        out_shape=[
            jax.ShapeDtypeStruct((PKM_H, S, PKM_KNN), jnp.float32),
            jax.ShapeDtypeStruct((PKM_H, S, PKM_KNN), jnp.int32),
        ],
    )(q, k1, k2)
    return w, i


_SC_CH = 32


def _sc_gather_weighted(values, x, idx, w):
    S, D = x.shape
    K = PKM_H * PKM_KNN
    nch = K // _SC_CH
    info = plsc.get_sparse_core_info()
    NC, NS, L = info.num_cores, info.num_subcores, info.num_lanes
    NW = NC * NS
    tpw = S // NW
    nj = D // L
    jb_w = 8
    njb = nj // jb_w

    mesh = plsc.VectorSubcoreMesh(core_axis_name="c", subcore_axis_name="s")

    @functools.partial(
        pl.kernel,
        out_type=jax.ShapeDtypeStruct((S, D), jnp.float32),
        mesh=mesh,
        scratch_types=[
            pltpu.VMEM((nch, _SC_CH), jnp.int32),
            pltpu.VMEM((K,), jnp.float32),
            pltpu.VMEM((_SC_CH, D), jnp.float32),
            pltpu.VMEM((_SC_CH, D), jnp.float32),
            pltpu.VMEM((D,), jnp.float32),
            pltpu.SemaphoreType.DMA,
            pltpu.SemaphoreType.DMA,
        ],
    )
    def sc_kernel(vals_hbm, x_hbm, idx_hbm, w_hbm, out_hbm,
                  idx_v, w_v, rows_a, rows_b, acc_v, sem_a, sem_b):
        wid = lax.axis_index("s") * NC + lax.axis_index("c")
        t0 = wid * tpw
        bufs = (rows_a, rows_b)
        sems = (sem_a, sem_b)

        def token_body(t, carry):
            tok = t0 + t
            pltpu.sync_copy(idx_hbm.at[tok], idx_v)
            pltpu.sync_copy(w_hbm.at[tok], w_v)
            pltpu.sync_copy(x_hbm.at[tok], acc_v)
            copies = [pltpu.async_copy(vals_hbm.at[idx_v.at[0]], bufs[0],
                                       sems[0])]
            for c in range(nch):
                copies[c].wait()
                if c + 1 < nch:
                    copies.append(pltpu.async_copy(
                        vals_hbm.at[idx_v.at[c + 1]],
                        bufs[(c + 1) % 2], sems[(c + 1) % 2]))
                buf = bufs[c % 2]
                for g in range(_SC_CH // L):
                    wv = w_v[pl.ds(c * _SC_CH + g * L, L)]
                    for jb in range(njb):
                        def rbody(r, acc, _buf=buf, _jb=jb, _g=g, _wv=wv):
                            wb = _wv.at[jnp.full((L,), r, jnp.int32)].get(
                                mode="promise_in_bounds")
                            return tuple(
                                acc[u] + wb * _buf[_g * L + r,
                                                   pl.ds(_jb * jb_w * L + u * L,
                                                         L)]
                                for u in range(jb_w))
                        acc8 = lax.fori_loop(
                            0, L, rbody,
                            tuple(jnp.zeros((L,), jnp.float32)
                                  for _ in range(jb_w)),
                            unroll=4)
                        for u in range(jb_w):
                            sl = pl.ds(jb * jb_w * L + u * L, L)
                            acc_v[sl] = acc_v[sl] + acc8[u]
            pltpu.sync_copy(acc_v, out_hbm.at[tok])
            return carry

        lax.fori_loop(0, tpw, token_body, 0)

    return sc_kernel(values, x, idx, w)


def _enc_layer(x, p, nheads):
    S, d = x.shape
    hd = d // nheads
    qkv = _matmul(x, p["Wqkv"], p["bqkv"])
    q = qkv[:, :d].reshape(S, nheads, hd).transpose(1, 0, 2)
    k = qkv[:, d:2 * d].reshape(S, nheads, hd).transpose(1, 0, 2)
    v = qkv[:, 2 * d:].reshape(S, nheads, hd).transpose(1, 0, 2)
    o = _attention(q, k, v)
    o = o.transpose(1, 0, 2).reshape(S, d)
    att = _matmul(o, p["Wo"], p["bo"])
    x = _res_ln(x, att, p["ln1g"], p["ln1b"])
    h = _matmul(x, p["W1"], p["b1"], relu=True)
    f = _matmul(h, p["W2"], p["b2"])
    return _res_ln(x, f, p["ln2g"], p["ln2b"])


def _pkm(x, p, nk):
    S, d = x.shape
    scale = p["bng"] / np.sqrt(1.0 + 1e-5)
    Wq = p["Wq"] * scale[:, None]
    bq = p["bq"] * scale + p["bnb"]
    q = _matmul(x, Wq, bq)
    w4, i4 = _pkm_score(q, p["keys"], nk)
    idx = i4.transpose(1, 0, 2).reshape(S, PKM_H * PKM_KNN // _SC_CH, _SC_CH)
    w = w4.transpose(1, 0, 2).reshape(S, PKM_H * PKM_KNN)
    return _sc_gather_weighted(p["values"], x, idx, w)


def kernel(x0, x1, params):
    S, B, D0 = x0.shape
    D1 = x1.shape[-1]
    a = x0.reshape(S, D0)
    b = x1.reshape(S, D1)
    for i in range(2):
        a = _enc_layer(a, params["enc0"][i], 16)
        b = _enc_layer(b, params["enc1"][i], 16)
        a = _pkm(a, params["mem0"], 168)
        b = _pkm(b, params["mem1"], 50)
    a = _res_ln(a, None, params["normg"], params["normb"])
    return (a.reshape(S, B, D0).transpose(1, 0, 2),
            b.reshape(S, B, D1).transpose(1, 0, 2))
